# Initial kernel scaffold; baseline (speedup 1.0000x reference)
import functools

import jax
import jax.numpy as jnp
from jax import lax
from jax.experimental import pallas as pl
from jax.experimental.pallas import tpu as pltpu
from jax.experimental.pallas import tpu_sc as plsc

_N = 10000
_E = 320000
_HID = 16

_IDXW = 128
_CHUNK = 512
_ROWS = _CHUNK // _IDXW
_NCH = _E // _CHUNK
_NW = 32
_NSUB = 16
_RPS = _N // _NSUB
_ACCW = 32
_MAXT = (_NCH + _NW - 1) // _NW


def _lrelu01(v):
    return jnp.maximum(v, 0.01 * v)


def _lrelu02(v):
    return jnp.maximum(v, 0.2 * v)


def _ln(v, g, b):
    mu = jnp.mean(v, -1, keepdims=True)
    d = v - mu
    var = jnp.mean(d * d, -1, keepdims=True)
    return d * lax.rsqrt(var + 1e-5) * g + b


def _dot(a, w):
    return jnp.dot(a, w, preferred_element_type=jnp.float32)


def _sc_layer_body(xl1_h, xr1_h, xl2_h, xr2_h, att_h,
                   src1_h, dst1_h, src2_h, dst2_h, out_h,
                   attv, srcv, dstv, gxl, gxr, outb, acc1, acc2):
    cid = lax.axis_index("c")
    sid = lax.axis_index("s")
    wid = sid * 2 + cid

    pltpu.sync_copy(att_h, attv)

    zero16 = jnp.zeros((16,), jnp.float32)

    def _zero_row(i, carry):
        outb[i, pl.ds(0, 16)] = zero16
        outb[i, pl.ds(16, 16)] = zero16
        return carry

    lax.fori_loop(0, _CHUNK, _zero_row, 0)

    base = sid * _RPS
    for acc in (acc1, acc2):
        pltpu.sync_copy(outb.at[pl.ds(0, _CHUNK)], acc.at[pl.ds(base, _CHUNK)])
        pltpu.sync_copy(outb.at[pl.ds(0, _RPS - _CHUNK)],
                        acc.at[pl.ds(base + _CHUNK, _RPS - _CHUNK)])
    plsc.subcore_barrier()

    iota16 = lax.broadcasted_iota(jnp.int32, (16,), 0)
    colids = [jnp.full((16,), f, jnp.int32) for f in range(_HID + 1)]

    convs = ((xl1_h, xr1_h, src1_h, dst1_h, acc1),
             (xl2_h, xr2_h, src2_h, dst2_h, acc2))
    for s, (xl_h, xr_h, src_h, dst_h, acc) in enumerate(convs):
        att_s = [attv[s, f] for f in range(_HID)]

        def _chunk(t, carry, xl_h=xl_h, xr_h=xr_h, src_h=src_h,
                   dst_h=dst_h, acc=acc, att_s=att_s):
            c = wid + t * _NW

            @pl.when(c < _NCH)
            def _():
                row0 = c * _ROWS
                pltpu.sync_copy(src_h.at[pl.ds(row0, _ROWS)], srcv)
                pltpu.sync_copy(dst_h.at[pl.ds(row0, _ROWS)], dstv)
                for r in range(_ROWS):
                    pltpu.sync_copy(xl_h.at[srcv.at[r]],
                                    gxl.at[pl.ds(r * _IDXW, _IDXW)])
                    pltpu.sync_copy(xr_h.at[dstv.at[r]],
                                    gxr.at[pl.ds(r * _IDXW, _IDXW)])

                def _group(g, gcarry):
                    rix = g * 16 + iota16
                    logit = jnp.zeros((16,), jnp.float32)
                    xlcols = []
                    for f in range(_HID):
                        a = plsc.load_gather(gxl, [rix, colids[f]])
                        b = plsc.load_gather(gxr, [rix, colids[f]])
                        e = _lrelu02(a + b)
                        logit = logit + att_s[f] * e
                        xlcols.append(a)
                    w = jnp.exp(logit)
                    for f in range(_HID):
                        plsc.store_scatter(outb, [rix, colids[f]], w * xlcols[f])
                    plsc.store_scatter(outb, [rix, colids[_HID]], w)
                    return gcarry

                lax.fori_loop(0, _CHUNK // 16, _group, 0)

                for r in range(_ROWS):
                    pltpu.sync_copy(outb.at[pl.ds(r * _IDXW, _IDXW)],
                                    acc.at[dstv.at[r]], add=True)

            return carry

        lax.fori_loop(0, _MAXT, _chunk, 0)

    plsc.subcore_barrier()

    for s, acc in ((0, acc1), (1, acc2)):
        off = (s * 2 + cid) * _N + base
        pltpu.sync_copy(acc.at[pl.ds(base, _RPS)], out_h.at[pl.ds(off, _RPS)])


_sc_layer = functools.partial(
    pl.kernel,
    out_type=jax.ShapeDtypeStruct((4 * _N, _ACCW), jnp.float32),
    mesh=plsc.VectorSubcoreMesh(core_axis_name="c", subcore_axis_name="s"),
    scratch_types=[
        pltpu.VMEM((2, _HID), jnp.float32),
        pltpu.VMEM((_ROWS, _IDXW), jnp.int32),
        pltpu.VMEM((_ROWS, _IDXW), jnp.int32),
        pltpu.VMEM((_CHUNK, _HID), jnp.float32),
        pltpu.VMEM((_CHUNK, _HID), jnp.float32),
        pltpu.VMEM((_CHUNK, _ACCW), jnp.float32),
        pltpu.VMEM_SHARED((_N, _ACCW), jnp.float32),
        pltpu.VMEM_SHARED((_N, _ACCW), jnp.float32),
    ],
)(_sc_layer_body)


def _tc_embed_body(x, w1, b1, w2, b2,
                   wl1, bl1, wr1, br1, wl2, bl2, wr2, br2,
                   h_o, xl1_o, xr1_o, xl2_o, xr2_o):
    h = _dot(_lrelu01(_dot(x[...], w1[...]) + b1[...]), w2[...]) + b2[...]
    h_o[...] = h
    xl1_o[...] = _dot(h, wl1[...]) + bl1[...]
    xr1_o[...] = _dot(h, wr1[...]) + br1[...]
    xl2_o[...] = _dot(h, wl2[...]) + bl2[...]
    xr2_o[...] = _dot(h, wr2[...]) + br2[...]


def _combine_convs(acc, h, xl1, xr1, xl2, xr2, att, bias1, bias2,
                   fcw, fcb, g1, bg1, l1w, l1b, l2w, l2b, g2, bg2):
    outs = []
    for s, (xl, xr, bias) in enumerate(((xl1, xr1, bias1), (xl2, xr2, bias2))):
        wself = jnp.exp(jnp.sum(_lrelu02(xl + xr) * att[s:s + 1, :],
                                axis=-1, keepdims=True))
        num = acc[2 * s, :, :_HID] + acc[2 * s + 1, :, :_HID] + wself * xl
        den = (acc[2 * s, :, _HID:_HID + 1] + acc[2 * s + 1, :, _HID:_HID + 1]
               + wself)
        outs.append(num / (den + 1e-16) + bias)
    hm = _dot(jnp.concatenate(outs, axis=-1), fcw) + fcb
    h1 = _ln(hm + h, g1, bg1)
    h2 = _dot(_lrelu01(_dot(h1, l1w) + l1b), l2w) + l2b
    return _ln(h2 + h1, g2, bg2)


def _tc_mid_body(acc, h, xl1, xr1, xl2, xr2, att, bias1, bias2,
                 fcw, fcb, g1, bg1, l1w, l1b, l2w, l2b, g2, bg2,
                 nwl1, nbl1, nwr1, nbr1, nwl2, nbl2, nwr2, nbr2,
                 h_o, xl1_o, xr1_o, xl2_o, xr2_o):
    hn = _combine_convs(acc[...], h[...], xl1[...], xr1[...], xl2[...],
                        xr2[...], att[...], bias1[...], bias2[...],
                        fcw[...], fcb[...], g1[...], bg1[...], l1w[...],
                        l1b[...], l2w[...], l2b[...], g2[...], bg2[...])
    h_o[...] = hn
    xl1_o[...] = _dot(hn, nwl1[...]) + nbl1[...]
    xr1_o[...] = _dot(hn, nwr1[...]) + nbr1[...]
    xl2_o[...] = _dot(hn, nwl2[...]) + nbl2[...]
    xr2_o[...] = _dot(hn, nwr2[...]) + nbr2[...]


def _tc_last_body(acc, h, xl1, xr1, xl2, xr2, att, bias1, bias2,
                  fcw, fcb, g1, bg1, l1w, l1b, l2w, l2b, g2, bg2,
                  f1w, f1b, f2w, f2b, f3w, f3b, out_o):
    hn = _combine_convs(acc[...], h[...], xl1[...], xr1[...], xl2[...],
                        xr2[...], att[...], bias1[...], bias2[...],
                        fcw[...], fcb[...], g1[...], bg1[...], l1w[...],
                        l1b[...], l2w[...], l2b[...], g2[...], bg2[...])
    y = _dot(_lrelu01(_dot(hn, f1w[...]) + f1b[...]), f2w[...]) + f2b[...]
    out_o[...] = _dot(_lrelu01(y), f3w[...]) + f3b[...]


def _nhid():
    return jax.ShapeDtypeStruct((_N, _HID), jnp.float32)


_tc_embed = pl.pallas_call(
    _tc_embed_body,
    out_shape=(_nhid(), _nhid(), _nhid(), _nhid(), _nhid()),
)

_tc_mid = pl.pallas_call(
    _tc_mid_body,
    out_shape=(_nhid(), _nhid(), _nhid(), _nhid(), _nhid()),
)

_tc_last = pl.pallas_call(
    _tc_last_body,
    out_shape=_nhid(),
)


def _r2(b):
    return b.reshape(1, -1)


def kernel(x, edge_index, global_edge_index, params):
    src1 = edge_index[0].astype(jnp.int32).reshape(_NCH * _ROWS, _IDXW)
    dst1 = edge_index[1].astype(jnp.int32).reshape(_NCH * _ROWS, _IDXW)
    src2 = global_edge_index[0].astype(jnp.int32).reshape(_NCH * _ROWS, _IDXW)
    dst2 = global_edge_index[1].astype(jnp.int32).reshape(_NCH * _ROWS, _IDXW)

    p = params
    emb = p["embed_fc"]

    def gatw(c):
        return (c["lin_l"]["W"], _r2(c["lin_l"]["b"]),
                c["lin_r"]["W"], _r2(c["lin_r"]["b"]))

    c0 = p["convs"][0]
    h, xl1, xr1, xl2, xr2 = _tc_embed(
        x, emb["l1"]["W"], _r2(emb["l1"]["b"]), emb["l2"]["W"],
        _r2(emb["l2"]["b"]),
        *gatw(c0["conv1"]), *gatw(c0["conv2"]))

    for i in range(5):
        conv = p["convs"][i]
        att = jnp.concatenate([conv["conv1"]["att"], conv["conv2"]["att"]], 0)
        acc = _sc_layer(xl1, xr1, xl2, xr2, att, src1, dst1, src2, dst2)
        acc = acc.reshape(4, _N, _ACCW)
        fcs = p["fcs"][i]
        common = (acc, h, xl1, xr1, xl2, xr2, att,
                  _r2(conv["conv1"]["bias"]), _r2(conv["conv2"]["bias"]),
                  conv["fc"]["W"], _r2(conv["fc"]["b"]),
                  _r2(p["gatnorms"][i]["g"]), _r2(p["gatnorms"][i]["b"]),
                  fcs["l1"]["W"], _r2(fcs["l1"]["b"]),
                  fcs["l2"]["W"], _r2(fcs["l2"]["b"]),
                  _r2(p["fcnorms"][i]["g"]), _r2(p["fcnorms"][i]["b"]))
        if i < 4:
            cn = p["convs"][i + 1]
            h, xl1, xr1, xl2, xr2 = _tc_mid(
                *common, *gatw(cn["conv1"]), *gatw(cn["conv2"]))
        else:
            f = p["fc_final"]
            out = _tc_last(
                *common, f["l1"]["W"], _r2(f["l1"]["b"]),
                f["l2"]["W"], _r2(f["l2"]["b"]),
                f["l3"]["W"], _r2(f["l3"]["b"]))
    return out


# speedup vs baseline: 27.6943x; 27.6943x over previous
import functools

import jax
import jax.numpy as jnp
from jax import lax
from jax.experimental import pallas as pl
from jax.experimental.pallas import tpu as pltpu
from jax.experimental.pallas import tpu_sc as plsc

_N = 10000
_E = 320000
_HID = 16

_IDXW = 128
_CHUNK = 512
_ROWS = _CHUNK // _IDXW
_NCH = _E // _CHUNK
_NW = 32
_NSUB = 16
_NITER = (_NCH + 2 * _NW - 1) // (2 * _NW)
_ACCW = 32
_ZPS = 624


def _lrelu01(v):
    return jnp.maximum(v, 0.01 * v)


def _lrelu02(v):
    return jnp.maximum(v, 0.2 * v)


def _ln(v, g, b):
    mu = jnp.mean(v, -1, keepdims=True)
    d = v - mu
    var = jnp.mean(d * d, -1, keepdims=True)
    return d * lax.rsqrt(var + 1e-5) * g + b


def _dot(a, w):
    return jnp.dot(a, w, preferred_element_type=jnp.float32)


def _sc_layer_body(xl1_h, xr1_h, xl2_h, xr2_h, att_h,
                   src1_h, dst1_h, src2_h, dst2_h, out_h,
                   attv, srcv, dstv, gxl, gxr, outb, acc1, acc2, sems):
    cid = lax.axis_index("c")
    sid = lax.axis_index("s")
    wid = sid * 2 + cid

    pltpu.sync_copy(att_h, attv)

    zero16 = jnp.zeros((16,), jnp.float32)

    def _zero_row(i, carry):
        for b in range(2):
            outb[b, i, pl.ds(0, 16)] = zero16
            outb[b, i, pl.ds(16, 16)] = zero16
        return carry

    lax.fori_loop(0, _CHUNK, _zero_row, 0)

    zbase = sid * _ZPS
    for acc in (acc1, acc2):
        pltpu.sync_copy(outb.at[0, pl.ds(0, 512)], acc.at[pl.ds(zbase, 512)])
        pltpu.sync_copy(outb.at[0, pl.ds(0, 112)],
                        acc.at[pl.ds(zbase + 512, 112)])

        @pl.when(sid == 0)
        def _(acc=acc):
            pltpu.sync_copy(outb.at[0, pl.ds(0, _N - _NSUB * _ZPS)],
                            acc.at[pl.ds(_NSUB * _ZPS, _N - _NSUB * _ZPS)])
    plsc.subcore_barrier()

    iota16 = lax.broadcasted_iota(jnp.int32, (16,), 0)
    colids = [jnp.full((16,), f, jnp.int32) for f in range(_HID + 1)]
    semI = (sems[0], sems[1])
    semG = (sems[2], sems[3])
    semS = (sems[4], sems[5])

    convs = ((xl1_h, xr1_h, src1_h, dst1_h, acc1, 0),
             (xl2_h, xr2_h, src2_h, dst2_h, acc2, 1))
    for (xl_h, xr_h, src_h, dst_h, acc, s) in convs:
        att_row = attv[pl.ds(s * _HID, _HID)]
        att_s = [att_row[f] for f in range(_HID)]

        def fire_idx(c, b, src_h=src_h, dst_h=dst_h):
            pltpu.async_copy(src_h.at[c], srcv.at[b], semI[b])
            pltpu.async_copy(dst_h.at[c], dstv.at[b], semI[b])

        def drain_idx(c, b, src_h=src_h, dst_h=dst_h):
            pltpu.make_async_copy(src_h.at[c], srcv.at[b], semI[b]).wait()
            pltpu.make_async_copy(dst_h.at[c], dstv.at[b], semI[b]).wait()

        def fire_gathers(b, xl_h=xl_h, xr_h=xr_h):
            for r in range(_ROWS):
                pltpu.async_copy(xl_h.at[srcv.at[b, r]],
                                 gxl.at[b, pl.ds(r * _IDXW, _IDXW)], semG[b])
                pltpu.async_copy(xr_h.at[dstv.at[b, r]],
                                 gxr.at[b, pl.ds(r * _IDXW, _IDXW)], semG[b])

        def drain_gathers(b, xl_h=xl_h):
            pltpu.make_async_copy(xl_h.at[pl.ds(0, _CHUNK)],
                                  gxl.at[b], semG[b]).wait()
            pltpu.make_async_copy(xl_h.at[pl.ds(0, _CHUNK)],
                                  gxr.at[b], semG[b]).wait()

        def fire_scatters(b, acc=acc):
            descs = []
            for r in range(_ROWS):
                descs.append(pltpu.async_copy(
                    outb.at[b, pl.ds(r * _IDXW, _IDXW)],
                    acc.at[dstv.at[b, r]], semS[b], add=True))
            return descs

        def compute(b, att_s=att_s):
            def _group(g, gcarry):
                rix = g * 16 + iota16
                logit = jnp.zeros((16,), jnp.float32)
                xlcols = []
                for f in range(_HID):
                    a = plsc.load_gather(gxl.at[b], [rix, colids[f]])
                    z = plsc.load_gather(gxr.at[b], [rix, colids[f]])
                    e = _lrelu02(a + z)
                    logit = logit + att_s[f] * e
                    xlcols.append(a)
                w = jnp.exp(logit)
                for f in range(_HID):
                    plsc.store_scatter(outb.at[b], [rix, colids[f]],
                                       w * xlcols[f])
                plsc.store_scatter(outb.at[b], [rix, colids[_HID]], w)
                return gcarry

            lax.fori_loop(0, _CHUNK // 16, _group, 0)

        fire_idx(wid, 0)
        drain_idx(wid, 0)
        fire_gathers(0)
        fire_idx(wid + _NW, 1)

        def _iter(t, carry):
            k0 = 2 * t
            c0 = wid + k0 * _NW
            c1 = c0 + _NW
            c2 = c1 + _NW
            c3 = c2 + _NW

            @pl.when(c0 < _NCH)
            def _():
                drain_gathers(0)

            @pl.when(c1 < _NCH)
            def _():
                drain_idx(c1, 1)
                fire_gathers(1)

            @pl.when(c0 < _NCH)
            def _():
                compute(0)
                descs = fire_scatters(0)
                for d in descs:
                    d.wait()

                @pl.when(c2 < _NCH)
                def _():
                    fire_idx(c2, 0)

            @pl.when(c1 < _NCH)
            def _():
                drain_gathers(1)

                @pl.when(c2 < _NCH)
                def _():
                    drain_idx(c2, 0)
                    fire_gathers(0)
                compute(1)
                descs = fire_scatters(1)
                for d in descs:
                    d.wait()

                @pl.when(c3 < _NCH)
                def _():
                    fire_idx(c3, 1)

            return carry

        lax.fori_loop(0, _NITER, _iter, 0)

    plsc.subcore_barrier()

    for s, acc in ((0, acc1), (1, acc2)):
        off = (s * 2 + cid) * _N + sid * _ZPS
        pltpu.sync_copy(acc.at[pl.ds(sid * _ZPS, _ZPS)],
                        out_h.at[pl.ds(off, _ZPS)])

        @pl.when(sid == 0)
        def _(acc=acc, s=s):
            tail = _N - _NSUB * _ZPS
            pltpu.sync_copy(acc.at[pl.ds(_NSUB * _ZPS, tail)],
                            out_h.at[pl.ds((s * 2 + cid) * _N
                                           + _NSUB * _ZPS, tail)])


_sc_layer = functools.partial(
    pl.kernel,
    out_type=jax.ShapeDtypeStruct((4 * _N, _ACCW), jnp.float32),
    mesh=plsc.VectorSubcoreMesh(core_axis_name="c", subcore_axis_name="s"),
    compiler_params=pltpu.CompilerParams(needs_layout_passes=False,
                                         use_tc_tiling_on_sc=False),
    scratch_types=[
        pltpu.VMEM((2 * _HID,), jnp.float32),
        pltpu.VMEM((2, _ROWS, _IDXW), jnp.int32),
        pltpu.VMEM((2, _ROWS, _IDXW), jnp.int32),
        pltpu.VMEM((2, _CHUNK, _HID), jnp.float32),
        pltpu.VMEM((2, _CHUNK, _HID), jnp.float32),
        pltpu.VMEM((2, _CHUNK, _ACCW), jnp.float32),
        pltpu.VMEM_SHARED((_N, _ACCW), jnp.float32),
        pltpu.VMEM_SHARED((_N, _ACCW), jnp.float32),
        [pltpu.SemaphoreType.DMA] * 6,
    ],
)(_sc_layer_body)


def _tc_embed_body(x, w1, b1, w2, b2,
                   wl1, bl1, wr1, br1, wl2, bl2, wr2, br2,
                   h_o, xl1_o, xr1_o, xl2_o, xr2_o):
    h = _dot(_lrelu01(_dot(x[...], w1[...]) + b1[...]), w2[...]) + b2[...]
    h_o[...] = h
    xl1_o[...] = _dot(h, wl1[...]) + bl1[...]
    xr1_o[...] = _dot(h, wr1[...]) + br1[...]
    xl2_o[...] = _dot(h, wl2[...]) + bl2[...]
    xr2_o[...] = _dot(h, wr2[...]) + br2[...]


def _combine_convs(acc, h, xl1, xr1, xl2, xr2, att, bias1, bias2,
                   fcw, fcb, g1, bg1, l1w, l1b, l2w, l2b, g2, bg2):
    outs = []
    for s, (xl, xr, bias) in enumerate(((xl1, xr1, bias1),
                                        (xl2, xr2, bias2))):
        wself = jnp.exp(jnp.sum(_lrelu02(xl + xr) * att[s:s + 1, :],
                                axis=-1, keepdims=True))
        num = acc[2 * s, :, :_HID] + acc[2 * s + 1, :, :_HID] + wself * xl
        den = (acc[2 * s, :, _HID:_HID + 1] + acc[2 * s + 1, :, _HID:_HID + 1]
               + wself)
        outs.append(num / (den + 1e-16) + bias)
    hm = _dot(jnp.concatenate(outs, axis=-1), fcw) + fcb
    h1 = _ln(hm + h, g1, bg1)
    h2 = _dot(_lrelu01(_dot(h1, l1w) + l1b), l2w) + l2b
    return _ln(h2 + h1, g2, bg2)


def _tc_mid_body(acc, h, xl1, xr1, xl2, xr2, att, bias1, bias2,
                 fcw, fcb, g1, bg1, l1w, l1b, l2w, l2b, g2, bg2,
                 nwl1, nbl1, nwr1, nbr1, nwl2, nbl2, nwr2, nbr2,
                 h_o, xl1_o, xr1_o, xl2_o, xr2_o):
    hn = _combine_convs(acc[...], h[...], xl1[...], xr1[...], xl2[...],
                        xr2[...], att[...], bias1[...], bias2[...],
                        fcw[...], fcb[...], g1[...], bg1[...], l1w[...],
                        l1b[...], l2w[...], l2b[...], g2[...], bg2[...])
    h_o[...] = hn
    xl1_o[...] = _dot(hn, nwl1[...]) + nbl1[...]
    xr1_o[...] = _dot(hn, nwr1[...]) + nbr1[...]
    xl2_o[...] = _dot(hn, nwl2[...]) + nbl2[...]
    xr2_o[...] = _dot(hn, nwr2[...]) + nbr2[...]


def _tc_last_body(acc, h, xl1, xr1, xl2, xr2, att, bias1, bias2,
                  fcw, fcb, g1, bg1, l1w, l1b, l2w, l2b, g2, bg2,
                  f1w, f1b, f2w, f2b, f3w, f3b, out_o):
    hn = _combine_convs(acc[...], h[...], xl1[...], xr1[...], xl2[...],
                        xr2[...], att[...], bias1[...], bias2[...],
                        fcw[...], fcb[...], g1[...], bg1[...], l1w[...],
                        l1b[...], l2w[...], l2b[...], g2[...], bg2[...])
    y = _dot(_lrelu01(_dot(hn, f1w[...]) + f1b[...]), f2w[...]) + f2b[...]
    out_o[...] = _dot(_lrelu01(y), f3w[...]) + f3b[...]


_TB = 1000
_TG = _N // _TB


def _shape_n():
    return jax.ShapeDtypeStruct((_N, _HID), jnp.float32)


def _rows_spec():
    return pl.BlockSpec((_TB, _HID), lambda i: (i, 0))


def _full_spec(shape):
    nd = len(shape)
    return pl.BlockSpec(shape, lambda i, nd=nd: (0,) * nd)


_tc_embed = pl.pallas_call(
    _tc_embed_body,
    grid=(_TG,),
    in_specs=[pl.BlockSpec((_TB, 128), lambda i: (i, 0))]
    + [_full_spec(s) for s in ((128, _HID), (1, _HID), (_HID, _HID),
                               (1, _HID))]
    + [_full_spec(s) for s in ((_HID, _HID), (1, _HID)) * 4],
    out_specs=tuple(_rows_spec() for _ in range(5)),
    out_shape=tuple(_shape_n() for _ in range(5)),
)

_mid_common_specs = (
    [pl.BlockSpec((4, _TB, _ACCW), lambda i: (0, i, 0))]
    + [_rows_spec() for _ in range(5)]
    + [_full_spec((2, _HID)), _full_spec((1, _HID)), _full_spec((1, _HID)),
       _full_spec((2 * _HID, _HID)), _full_spec((1, _HID)),
       _full_spec((1, _HID)), _full_spec((1, _HID)),
       _full_spec((_HID, _HID)), _full_spec((1, _HID)),
       _full_spec((_HID, _HID)), _full_spec((1, _HID)),
       _full_spec((1, _HID)), _full_spec((1, _HID))]
)

_tc_mid = pl.pallas_call(
    _tc_mid_body,
    grid=(_TG,),
    in_specs=_mid_common_specs
    + [_full_spec(s) for s in ((_HID, _HID), (1, _HID)) * 4],
    out_specs=tuple(_rows_spec() for _ in range(5)),
    out_shape=tuple(_shape_n() for _ in range(5)),
)

_tc_last = pl.pallas_call(
    _tc_last_body,
    grid=(_TG,),
    in_specs=_mid_common_specs
    + [_full_spec(s) for s in ((_HID, _HID), (1, _HID)) * 3],
    out_specs=_rows_spec(),
    out_shape=_shape_n(),
)


def _r2(b):
    return b.reshape(1, -1)


def _edge3d(v):
    return v.astype(jnp.int32).reshape(_NCH, _ROWS, _IDXW)


def kernel(x, edge_index, global_edge_index, params):
    src1 = _edge3d(edge_index[0])
    dst1 = _edge3d(edge_index[1])
    src2 = _edge3d(global_edge_index[0])
    dst2 = _edge3d(global_edge_index[1])

    p = params
    emb = p["embed_fc"]

    def gatw(c):
        return (c["lin_l"]["W"], _r2(c["lin_l"]["b"]),
                c["lin_r"]["W"], _r2(c["lin_r"]["b"]))

    c0 = p["convs"][0]
    h, xl1, xr1, xl2, xr2 = _tc_embed(
        x, emb["l1"]["W"], _r2(emb["l1"]["b"]), emb["l2"]["W"],
        _r2(emb["l2"]["b"]),
        *gatw(c0["conv1"]), *gatw(c0["conv2"]))

    for i in range(5):
        conv = p["convs"][i]
        att = jnp.concatenate([conv["conv1"]["att"], conv["conv2"]["att"]], 0)
        acc = _sc_layer(xl1, xr1, xl2, xr2, att.reshape(2 * _HID),
                        src1, dst1, src2, dst2)
        acc = acc.reshape(4, _N, _ACCW)
        fcs = p["fcs"][i]
        common = (acc, h, xl1, xr1, xl2, xr2, att,
                  _r2(conv["conv1"]["bias"]), _r2(conv["conv2"]["bias"]),
                  conv["fc"]["W"], _r2(conv["fc"]["b"]),
                  _r2(p["gatnorms"][i]["g"]), _r2(p["gatnorms"][i]["b"]),
                  fcs["l1"]["W"], _r2(fcs["l1"]["b"]),
                  fcs["l2"]["W"], _r2(fcs["l2"]["b"]),
                  _r2(p["fcnorms"][i]["g"]), _r2(p["fcnorms"][i]["b"]))
        if i < 4:
            cn = p["convs"][i + 1]
            h, xl1, xr1, xl2, xr2 = _tc_mid(
                *common, *gatw(cn["conv1"]), *gatw(cn["conv2"]))
        else:
            f = p["fc_final"]
            out = _tc_last(
                *common, f["l1"]["W"], _r2(f["l1"]["b"]),
                f["l2"]["W"], _r2(f["l2"]["b"]),
                f["l3"]["W"], _r2(f["l3"]["b"]))
    return out


# speedup vs baseline: 28.5773x; 1.0319x over previous
import functools

import jax
import jax.numpy as jnp
from jax import lax
from jax.experimental import pallas as pl
from jax.experimental.pallas import tpu as pltpu
from jax.experimental.pallas import tpu_sc as plsc

_N = 10000
_E = 320000
_HID = 16

_IDXW = 128
_CHUNK = 512
_ROWS = _CHUNK // _IDXW
_NCH = _E // _CHUNK
_NW = 32
_NSUB = 16
_NITER = (_NCH + 2 * _NW - 1) // (2 * _NW)
_ACCW = 32
_ZPS = 624


def _lrelu01(v):
    return jnp.maximum(v, 0.01 * v)


def _lrelu02(v):
    return jnp.maximum(v, 0.2 * v)


def _ln(v, g, b):
    mu = jnp.mean(v, -1, keepdims=True)
    d = v - mu
    var = jnp.mean(d * d, -1, keepdims=True)
    return d * lax.rsqrt(var + 1e-5) * g + b


def _dot(a, w):
    return jnp.dot(a, w, preferred_element_type=jnp.float32)


def _sc_layer_body(xl1_h, xr1_h, xl2_h, xr2_h, att_h,
                   src1_h, dst1_h, src2_h, dst2_h, out_h,
                   attv, srcv, dstv, scat, gxl, gxr, outb, acc1, acc2, sems):
    cid = lax.axis_index("c")
    sid = lax.axis_index("s")
    wid = sid * 2 + cid

    pltpu.sync_copy(att_h, attv)

    zero16 = jnp.zeros((16,), jnp.float32)

    def _zero_row(i, carry):
        for b in range(2):
            outb[b, i, pl.ds(0, 16)] = zero16
            outb[b, i, pl.ds(16, 16)] = zero16
        return carry

    lax.fori_loop(0, _CHUNK, _zero_row, 0)

    zbase = sid * _ZPS
    for acc in (acc1, acc2):
        pltpu.sync_copy(outb.at[0, pl.ds(0, 512)], acc.at[pl.ds(zbase, 512)])
        pltpu.sync_copy(outb.at[0, pl.ds(0, 112)],
                        acc.at[pl.ds(zbase + 512, 112)])

        @pl.when(sid == 0)
        def _(acc=acc):
            pltpu.sync_copy(outb.at[0, pl.ds(0, _N - _NSUB * _ZPS)],
                            acc.at[pl.ds(_NSUB * _ZPS, _N - _NSUB * _ZPS)])
    plsc.subcore_barrier()

    iota16 = lax.broadcasted_iota(jnp.int32, (16,), 0)
    colids = [jnp.full((16,), f, jnp.int32) for f in range(_HID + 1)]
    semI = (sems[0], sems[1])
    semG = (sems[2], sems[3])
    semS = (sems[4], sems[5])

    convs = ((xl1_h, xr1_h, src1_h, dst1_h, acc1, 0),
             (xl2_h, xr2_h, src2_h, dst2_h, acc2, 1))
    for (xl_h, xr_h, src_h, dst_h, acc, s) in convs:
        att_row = attv[pl.ds(s * _HID, _HID)]
        att_s = [att_row[f] for f in range(_HID)]

        def fire_idx(c, b, src_h=src_h, dst_h=dst_h):
            pltpu.async_copy(src_h.at[c], srcv.at[b], semI[b])
            pltpu.async_copy(dst_h.at[c], dstv.at[b], semI[b])

        def drain_idx(c, b, src_h=src_h, dst_h=dst_h):
            pltpu.make_async_copy(src_h.at[c], srcv.at[b], semI[b]).wait()
            pltpu.make_async_copy(dst_h.at[c], dstv.at[b], semI[b]).wait()

        def fire_gathers(b, xl_h=xl_h, xr_h=xr_h):
            for r in range(_ROWS):
                pltpu.async_copy(xl_h.at[srcv.at[b, r]],
                                 gxl.at[b, pl.ds(r * _IDXW, _IDXW)], semG[b])
                pltpu.async_copy(xr_h.at[dstv.at[b, r]],
                                 gxr.at[b, pl.ds(r * _IDXW, _IDXW)], semG[b])

        def drain_gathers(b, xl_h=xl_h):
            pltpu.make_async_copy(xl_h.at[pl.ds(0, _CHUNK)],
                                  gxl.at[b], semG[b]).wait()
            pltpu.make_async_copy(xl_h.at[pl.ds(0, _CHUNK)],
                                  gxr.at[b], semG[b]).wait()

        def copy_scat(b):
            for r in range(_ROWS):
                for j in range(_IDXW // 16):
                    scat[b, r, pl.ds(j * 16, 16)] = \
                        dstv[b, r, pl.ds(j * 16, 16)]

        def fire_scatters(b, acc=acc):
            for r in range(_ROWS):
                pltpu.async_copy(outb.at[b, pl.ds(r * _IDXW, _IDXW)],
                                 acc.at[scat.at[b, r]], semS[b], add=True)

        def drain_scatters(b, acc=acc):
            for r in range(_ROWS):
                pltpu.make_async_copy(outb.at[b, pl.ds(r * _IDXW, _IDXW)],
                                      acc.at[scat.at[b, r]],
                                      semS[b]).wait()

        def compute(b, att_s=att_s):
            def _group(g, gcarry):
                for u in range(2):
                    rix = (g * 2 + u) * 16 + iota16
                    xlcols = []
                    terms = []
                    for f in range(_HID):
                        a = plsc.load_gather(gxl.at[b], [rix, colids[f]])
                        z = plsc.load_gather(gxr.at[b], [rix, colids[f]])
                        terms.append(att_s[f] * _lrelu02(a + z))
                        xlcols.append(a)
                    while len(terms) > 1:
                        terms = [terms[k] + terms[k + 1]
                                 for k in range(0, len(terms), 2)]
                    w = jnp.exp(terms[0])
                    for f in range(_HID):
                        plsc.store_scatter(outb.at[b], [rix, colids[f]],
                                           w * xlcols[f])
                    plsc.store_scatter(outb.at[b], [rix, colids[_HID]], w)
                return gcarry

            lax.fori_loop(0, _CHUNK // 32, _group, 0)

        fire_idx(wid, 0)
        drain_idx(wid, 0)
        fire_gathers(0)
        fire_idx(wid + _NW, 1)

        def _iter(t, carry):
            k0 = 2 * t
            c0 = wid + k0 * _NW
            c1 = c0 + _NW
            c2 = c1 + _NW
            c3 = c2 + _NW

            @pl.when(c0 < _NCH)
            def _():
                drain_gathers(0)

            @pl.when(c1 < _NCH)
            def _():
                drain_idx(c1, 1)
                fire_gathers(1)

            @pl.when(c0 < _NCH)
            def _():
                @pl.when(c0 >= 2 * _NW)
                def _():
                    drain_scatters(0)
                copy_scat(0)
                compute(0)
                fire_scatters(0)

                @pl.when(c2 < _NCH)
                def _():
                    fire_idx(c2, 0)

            @pl.when(c1 < _NCH)
            def _():
                drain_gathers(1)

                @pl.when(c2 < _NCH)
                def _():
                    drain_idx(c2, 0)
                    fire_gathers(0)

                @pl.when(c1 >= 2 * _NW)
                def _():
                    drain_scatters(1)
                copy_scat(1)
                compute(1)
                fire_scatters(1)

                @pl.when(c3 < _NCH)
                def _():
                    fire_idx(c3, 1)

            return carry

        lax.fori_loop(0, _NITER, _iter, 0)
        drain_scatters(0)
        drain_scatters(1)

    plsc.subcore_barrier()

    for s, acc in ((0, acc1), (1, acc2)):
        off = (s * 2 + cid) * _N + sid * _ZPS
        pltpu.sync_copy(acc.at[pl.ds(sid * _ZPS, _ZPS)],
                        out_h.at[pl.ds(off, _ZPS)])

        @pl.when(sid == 0)
        def _(acc=acc, s=s):
            tail = _N - _NSUB * _ZPS
            pltpu.sync_copy(acc.at[pl.ds(_NSUB * _ZPS, tail)],
                            out_h.at[pl.ds((s * 2 + cid) * _N
                                           + _NSUB * _ZPS, tail)])


_sc_layer = functools.partial(
    pl.kernel,
    out_type=jax.ShapeDtypeStruct((4 * _N, _ACCW), jnp.float32),
    mesh=plsc.VectorSubcoreMesh(core_axis_name="c", subcore_axis_name="s"),
    compiler_params=pltpu.CompilerParams(needs_layout_passes=False,
                                         use_tc_tiling_on_sc=False),
    scratch_types=[
        pltpu.VMEM((2 * _HID,), jnp.float32),
        pltpu.VMEM((2, _ROWS, _IDXW), jnp.int32),
        pltpu.VMEM((2, _ROWS, _IDXW), jnp.int32),
        pltpu.VMEM((2, _ROWS, _IDXW), jnp.int32),
        pltpu.VMEM((2, _CHUNK, _HID), jnp.float32),
        pltpu.VMEM((2, _CHUNK, _HID), jnp.float32),
        pltpu.VMEM((2, _CHUNK, _ACCW), jnp.float32),
        pltpu.VMEM_SHARED((_N, _ACCW), jnp.float32),
        pltpu.VMEM_SHARED((_N, _ACCW), jnp.float32),
        [pltpu.SemaphoreType.DMA] * 6,
    ],
)(_sc_layer_body)


def _tc_embed_body(x, w1, b1, w2, b2,
                   wl1, bl1, wr1, br1, wl2, bl2, wr2, br2,
                   h_o, xl1_o, xr1_o, xl2_o, xr2_o):
    h = _dot(_lrelu01(_dot(x[...], w1[...]) + b1[...]), w2[...]) + b2[...]
    h_o[...] = h
    xl1_o[...] = _dot(h, wl1[...]) + bl1[...]
    xr1_o[...] = _dot(h, wr1[...]) + br1[...]
    xl2_o[...] = _dot(h, wl2[...]) + bl2[...]
    xr2_o[...] = _dot(h, wr2[...]) + br2[...]


def _combine_convs(acc, h, xl1, xr1, xl2, xr2, att, bias1, bias2,
                   fcw, fcb, g1, bg1, l1w, l1b, l2w, l2b, g2, bg2):
    outs = []
    for s, (xl, xr, bias) in enumerate(((xl1, xr1, bias1),
                                        (xl2, xr2, bias2))):
        wself = jnp.exp(jnp.sum(_lrelu02(xl + xr) * att[s:s + 1, :],
                                axis=-1, keepdims=True))
        num = acc[2 * s, :, :_HID] + acc[2 * s + 1, :, :_HID] + wself * xl
        den = (acc[2 * s, :, _HID:_HID + 1] + acc[2 * s + 1, :, _HID:_HID + 1]
               + wself)
        outs.append(num / (den + 1e-16) + bias)
    hm = _dot(jnp.concatenate(outs, axis=-1), fcw) + fcb
    h1 = _ln(hm + h, g1, bg1)
    h2 = _dot(_lrelu01(_dot(h1, l1w) + l1b), l2w) + l2b
    return _ln(h2 + h1, g2, bg2)


def _tc_mid_body(acc, h, xl1, xr1, xl2, xr2, att, bias1, bias2,
                 fcw, fcb, g1, bg1, l1w, l1b, l2w, l2b, g2, bg2,
                 nwl1, nbl1, nwr1, nbr1, nwl2, nbl2, nwr2, nbr2,
                 h_o, xl1_o, xr1_o, xl2_o, xr2_o):
    hn = _combine_convs(acc[...], h[...], xl1[...], xr1[...], xl2[...],
                        xr2[...], att[...], bias1[...], bias2[...],
                        fcw[...], fcb[...], g1[...], bg1[...], l1w[...],
                        l1b[...], l2w[...], l2b[...], g2[...], bg2[...])
    h_o[...] = hn
    xl1_o[...] = _dot(hn, nwl1[...]) + nbl1[...]
    xr1_o[...] = _dot(hn, nwr1[...]) + nbr1[...]
    xl2_o[...] = _dot(hn, nwl2[...]) + nbl2[...]
    xr2_o[...] = _dot(hn, nwr2[...]) + nbr2[...]


def _tc_last_body(acc, h, xl1, xr1, xl2, xr2, att, bias1, bias2,
                  fcw, fcb, g1, bg1, l1w, l1b, l2w, l2b, g2, bg2,
                  f1w, f1b, f2w, f2b, f3w, f3b, out_o):
    hn = _combine_convs(acc[...], h[...], xl1[...], xr1[...], xl2[...],
                        xr2[...], att[...], bias1[...], bias2[...],
                        fcw[...], fcb[...], g1[...], bg1[...], l1w[...],
                        l1b[...], l2w[...], l2b[...], g2[...], bg2[...])
    y = _dot(_lrelu01(_dot(hn, f1w[...]) + f1b[...]), f2w[...]) + f2b[...]
    out_o[...] = _dot(_lrelu01(y), f3w[...]) + f3b[...]


_TB = 1000
_TG = _N // _TB


def _shape_n():
    return jax.ShapeDtypeStruct((_N, _HID), jnp.float32)


def _rows_spec():
    return pl.BlockSpec((_TB, _HID), lambda i: (i, 0))


def _full_spec(shape):
    nd = len(shape)
    return pl.BlockSpec(shape, lambda i, nd=nd: (0,) * nd)


_tc_embed = pl.pallas_call(
    _tc_embed_body,
    grid=(_TG,),
    in_specs=[pl.BlockSpec((_TB, 128), lambda i: (i, 0))]
    + [_full_spec(s) for s in ((128, _HID), (1, _HID), (_HID, _HID),
                               (1, _HID))]
    + [_full_spec(s) for s in ((_HID, _HID), (1, _HID)) * 4],
    out_specs=tuple(_rows_spec() for _ in range(5)),
    out_shape=tuple(_shape_n() for _ in range(5)),
)

_mid_common_specs = (
    [pl.BlockSpec((4, _TB, _ACCW), lambda i: (0, i, 0))]
    + [_rows_spec() for _ in range(5)]
    + [_full_spec((2, _HID)), _full_spec((1, _HID)), _full_spec((1, _HID)),
       _full_spec((2 * _HID, _HID)), _full_spec((1, _HID)),
       _full_spec((1, _HID)), _full_spec((1, _HID)),
       _full_spec((_HID, _HID)), _full_spec((1, _HID)),
       _full_spec((_HID, _HID)), _full_spec((1, _HID)),
       _full_spec((1, _HID)), _full_spec((1, _HID))]
)

_tc_mid = pl.pallas_call(
    _tc_mid_body,
    grid=(_TG,),
    in_specs=_mid_common_specs
    + [_full_spec(s) for s in ((_HID, _HID), (1, _HID)) * 4],
    out_specs=tuple(_rows_spec() for _ in range(5)),
    out_shape=tuple(_shape_n() for _ in range(5)),
)

_tc_last = pl.pallas_call(
    _tc_last_body,
    grid=(_TG,),
    in_specs=_mid_common_specs
    + [_full_spec(s) for s in ((_HID, _HID), (1, _HID)) * 3],
    out_specs=_rows_spec(),
    out_shape=_shape_n(),
)


def _r2(b):
    return b.reshape(1, -1)


def _edge3d(v):
    return v.astype(jnp.int32).reshape(_NCH, _ROWS, _IDXW)


def kernel(x, edge_index, global_edge_index, params):
    src1 = _edge3d(edge_index[0])
    dst1 = _edge3d(edge_index[1])
    src2 = _edge3d(global_edge_index[0])
    dst2 = _edge3d(global_edge_index[1])

    p = params
    emb = p["embed_fc"]

    def gatw(c):
        return (c["lin_l"]["W"], _r2(c["lin_l"]["b"]),
                c["lin_r"]["W"], _r2(c["lin_r"]["b"]))

    c0 = p["convs"][0]
    h, xl1, xr1, xl2, xr2 = _tc_embed(
        x, emb["l1"]["W"], _r2(emb["l1"]["b"]), emb["l2"]["W"],
        _r2(emb["l2"]["b"]),
        *gatw(c0["conv1"]), *gatw(c0["conv2"]))

    for i in range(5):
        conv = p["convs"][i]
        att = jnp.concatenate([conv["conv1"]["att"], conv["conv2"]["att"]], 0)
        acc = _sc_layer(xl1, xr1, xl2, xr2, att.reshape(2 * _HID),
                        src1, dst1, src2, dst2)
        acc = acc.reshape(4, _N, _ACCW)
        fcs = p["fcs"][i]
        common = (acc, h, xl1, xr1, xl2, xr2, att,
                  _r2(conv["conv1"]["bias"]), _r2(conv["conv2"]["bias"]),
                  conv["fc"]["W"], _r2(conv["fc"]["b"]),
                  _r2(p["gatnorms"][i]["g"]), _r2(p["gatnorms"][i]["b"]),
                  fcs["l1"]["W"], _r2(fcs["l1"]["b"]),
                  fcs["l2"]["W"], _r2(fcs["l2"]["b"]),
                  _r2(p["fcnorms"][i]["g"]), _r2(p["fcnorms"][i]["b"]))
        if i < 4:
            cn = p["convs"][i + 1]
            h, xl1, xr1, xl2, xr2 = _tc_mid(
                *common, *gatw(cn["conv1"]), *gatw(cn["conv2"]))
        else:
            f = p["fc_final"]
            out = _tc_last(
                *common, f["l1"]["W"], _r2(f["l1"]["b"]),
                f["l2"]["W"], _r2(f["l2"]["b"]),
                f["l3"]["W"], _r2(f["l3"]["b"]))
    return out


# speedup vs baseline: 43.7797x; 1.5320x over previous
import functools

import jax
import jax.numpy as jnp
from jax import lax
from jax.experimental import pallas as pl
from jax.experimental.pallas import tpu as pltpu
from jax.experimental.pallas import tpu_sc as plsc

_N = 10000
_E = 320000
_HID = 16

_IDXW = 128
_CHUNK = 512
_ROWS = _CHUNK // _IDXW
_NCH = _E // _CHUNK
_NW = 32
_NSUB = 16
_NITER = (_NCH + 2 * _NW - 1) // (2 * _NW)
_ACCW = 32
_ZPS = 624


def _lrelu01(v):
    return jnp.maximum(v, 0.01 * v)


def _lrelu02(v):
    return jnp.maximum(v, 0.2 * v)


def _ln(v, g, b):
    mu = jnp.mean(v, -1, keepdims=True)
    d = v - mu
    var = jnp.mean(d * d, -1, keepdims=True)
    return d * lax.rsqrt(var + 1e-5) * g + b


def _dot(a, w):
    return jnp.dot(a, w, preferred_element_type=jnp.float32)


def _sc_layer_body(xl1_h, xr1_h, xl2_h, xr2_h, att_h,
                   src1_h, dst1_h, src2_h, dst2_h, out_h, den_h,
                   attv, srcv, dstv, scat, gxl, gxr, outb, dent,
                   acc1, acc2, sems):
    cid = lax.axis_index("c")
    sid = lax.axis_index("s")
    wid = sid * 2 + cid

    pltpu.sync_copy(att_h, attv)

    zero16 = jnp.zeros((16,), jnp.float32)

    def _zero_row(i, carry):
        for b in range(2):
            outb[b, i, pl.ds(0, 16)] = zero16
        return carry

    lax.fori_loop(0, _CHUNK, _zero_row, 0)

    def _zero_den(i, carry):
        dent[pl.ds(i * 16, 16)] = zero16
        return carry

    zbase = sid * _ZPS
    for acc in (acc1, acc2):
        pltpu.sync_copy(outb.at[0, pl.ds(0, 512)], acc.at[pl.ds(zbase, 512)])
        pltpu.sync_copy(outb.at[0, pl.ds(0, 112)],
                        acc.at[pl.ds(zbase + 512, 112)])

        @pl.when(sid == 0)
        def _(acc=acc):
            pltpu.sync_copy(outb.at[0, pl.ds(0, _N - _NSUB * _ZPS)],
                            acc.at[pl.ds(_NSUB * _ZPS, _N - _NSUB * _ZPS)])
    plsc.subcore_barrier()

    iota16 = lax.broadcasted_iota(jnp.int32, (16,), 0)
    colids = [jnp.full((16,), f, jnp.int32) for f in range(_HID + 1)]
    semI = (sems[0], sems[1])
    semG = (sems[2], sems[3])
    semS = (sems[4], sems[5])

    convs = ((xl1_h, xr1_h, src1_h, dst1_h, acc1, 0),
             (xl2_h, xr2_h, src2_h, dst2_h, acc2, 1))
    for (xl_h, xr_h, src_h, dst_h, acc, s) in convs:
        att_row = attv[pl.ds(s * _HID, _HID)]
        att_s = [att_row[f] for f in range(_HID)]
        lax.fori_loop(0, _N // 16, _zero_den, 0)

        def fire_idx(c, b, src_h=src_h, dst_h=dst_h):
            pltpu.async_copy(src_h.at[c], srcv.at[b], semI[b])
            pltpu.async_copy(dst_h.at[c], dstv.at[b], semI[b])

        def drain_idx(c, b, src_h=src_h, dst_h=dst_h):
            pltpu.make_async_copy(src_h.at[c], srcv.at[b], semI[b]).wait()
            pltpu.make_async_copy(dst_h.at[c], dstv.at[b], semI[b]).wait()

        def fire_gathers(b, xl_h=xl_h, xr_h=xr_h):
            for r in range(_ROWS):
                pltpu.async_copy(xl_h.at[srcv.at[b, r]],
                                 gxl.at[b, pl.ds(r * _IDXW, _IDXW)], semG[b])
                pltpu.async_copy(xr_h.at[dstv.at[b, r]],
                                 gxr.at[b, pl.ds(r * _IDXW, _IDXW)], semG[b])

        def drain_gathers(b, xl_h=xl_h):
            pltpu.make_async_copy(xl_h.at[pl.ds(0, _CHUNK)],
                                  gxl.at[b], semG[b]).wait()
            pltpu.make_async_copy(xl_h.at[pl.ds(0, _CHUNK)],
                                  gxr.at[b], semG[b]).wait()

        def copy_scat(b):
            for r in range(_ROWS):
                for j in range(_IDXW // 16):
                    scat[b, r, pl.ds(j * 16, 16)] = \
                        dstv[b, r, pl.ds(j * 16, 16)]

        def fire_scatters(b, acc=acc):
            for r in range(_ROWS):
                pltpu.async_copy(outb.at[b, pl.ds(r * _IDXW, _IDXW)],
                                 acc.at[scat.at[b, r]], semS[b], add=True)

        def drain_scatters(b, acc=acc):
            for r in range(_ROWS):
                pltpu.make_async_copy(outb.at[b, pl.ds(r * _IDXW, _IDXW)],
                                      acc.at[scat.at[b, r]],
                                      semS[b]).wait()

        def compute(b, att_s=att_s):
            def _group(g, gcarry):
                for u in range(2):
                    gg = g * 2 + u
                    rix = gg * 16 + iota16
                    xlcols = []
                    terms = []
                    for f in range(_HID):
                        a = plsc.load_gather(gxl.at[b], [rix, colids[f]])
                        z = plsc.load_gather(gxr.at[b], [rix, colids[f]])
                        terms.append(att_s[f] * _lrelu02(a + z))
                        xlcols.append(a)
                    while len(terms) > 1:
                        terms = [terms[k] + terms[k + 1]
                                 for k in range(0, len(terms), 2)]
                    w = jnp.exp(terms[0])
                    for f in range(_HID):
                        plsc.store_scatter(outb.at[b], [rix, colids[f]],
                                           w * xlcols[f])
                    dvec = dstv[b, gg // 8, pl.ds((gg % 8) * 16, 16)]
                    plsc.addupdate_scatter(dent, [dvec], w)
                return gcarry

            lax.fori_loop(0, _CHUNK // 32, _group, 0)

        fire_idx(wid, 0)
        drain_idx(wid, 0)
        fire_gathers(0)
        fire_idx(wid + _NW, 1)

        def _iter(t, carry):
            k0 = 2 * t
            c0 = wid + k0 * _NW
            c1 = c0 + _NW
            c2 = c1 + _NW
            c3 = c2 + _NW

            @pl.when(c0 < _NCH)
            def _():
                drain_gathers(0)

            @pl.when(c1 < _NCH)
            def _():
                drain_idx(c1, 1)
                fire_gathers(1)

            @pl.when(c0 < _NCH)
            def _():
                @pl.when(c0 >= 2 * _NW)
                def _():
                    drain_scatters(0)
                copy_scat(0)
                compute(0)
                fire_scatters(0)

                @pl.when(c2 < _NCH)
                def _():
                    fire_idx(c2, 0)

            @pl.when(c1 < _NCH)
            def _():
                drain_gathers(1)

                @pl.when(c2 < _NCH)
                def _():
                    drain_idx(c2, 0)
                    fire_gathers(0)

                @pl.when(c1 >= 2 * _NW)
                def _():
                    drain_scatters(1)
                copy_scat(1)
                compute(1)
                fire_scatters(1)

                @pl.when(c3 < _NCH)
                def _():
                    fire_idx(c3, 1)

            return carry

        lax.fori_loop(0, _NITER, _iter, 0)
        drain_scatters(0)
        drain_scatters(1)
        pltpu.sync_copy(dent, den_h.at[s * _NW + wid])

    plsc.subcore_barrier()

    for s, acc in ((0, acc1), (1, acc2)):
        off = (s * 2 + cid) * _N + sid * _ZPS
        pltpu.sync_copy(acc.at[pl.ds(sid * _ZPS, _ZPS)],
                        out_h.at[pl.ds(off, _ZPS)])

        @pl.when(sid == 0)
        def _(acc=acc, s=s):
            tail = _N - _NSUB * _ZPS
            pltpu.sync_copy(acc.at[pl.ds(_NSUB * _ZPS, tail)],
                            out_h.at[pl.ds((s * 2 + cid) * _N
                                           + _NSUB * _ZPS, tail)])


_sc_layer = functools.partial(
    pl.kernel,
    out_type=(jax.ShapeDtypeStruct((4 * _N, _HID), jnp.float32),
              jax.ShapeDtypeStruct((2 * _NW, _N), jnp.float32)),
    mesh=plsc.VectorSubcoreMesh(core_axis_name="c", subcore_axis_name="s"),
    compiler_params=pltpu.CompilerParams(needs_layout_passes=False,
                                         use_tc_tiling_on_sc=False),
    scratch_types=[
        pltpu.VMEM((2 * _HID,), jnp.float32),
        pltpu.VMEM((2, _ROWS, _IDXW), jnp.int32),
        pltpu.VMEM((2, _ROWS, _IDXW), jnp.int32),
        pltpu.VMEM((2, _ROWS, _IDXW), jnp.int32),
        pltpu.VMEM((2, _CHUNK, _HID), jnp.float32),
        pltpu.VMEM((2, _CHUNK, _HID), jnp.float32),
        pltpu.VMEM((2, _CHUNK, _HID), jnp.float32),
        pltpu.VMEM((_N,), jnp.float32),
        pltpu.VMEM_SHARED((_N, _HID), jnp.float32),
        pltpu.VMEM_SHARED((_N, _HID), jnp.float32),
        [pltpu.SemaphoreType.DMA] * 6,
    ],
)(_sc_layer_body)


def _tc_embed_body(x, w1, b1, w2, b2,
                   wl1, bl1, wr1, br1, wl2, bl2, wr2, br2,
                   h_o, xl1_o, xr1_o, xl2_o, xr2_o):
    h = _dot(_lrelu01(_dot(x[...], w1[...]) + b1[...]), w2[...]) + b2[...]
    h_o[...] = h
    xl1_o[...] = _dot(h, wl1[...]) + bl1[...]
    xr1_o[...] = _dot(h, wr1[...]) + br1[...]
    xl2_o[...] = _dot(h, wl2[...]) + bl2[...]
    xr2_o[...] = _dot(h, wr2[...]) + br2[...]


def _combine_convs(acc, denp, h, xl1, xr1, xl2, xr2, att, bias1, bias2,
                   fcw, fcb, g1, bg1, l1w, l1b, l2w, l2b, g2, bg2):
    outs = []
    for s, (xl, xr, bias) in enumerate(((xl1, xr1, bias1),
                                        (xl2, xr2, bias2))):
        wself = jnp.exp(jnp.sum(_lrelu02(xl + xr) * att[s:s + 1, :],
                                axis=-1, keepdims=True))
        num = acc[2 * s] + acc[2 * s + 1] + wself * xl
        den = jnp.sum(denp[s], axis=0)[:, None] + wself
        outs.append(num / (den + 1e-16) + bias)
    hm = _dot(jnp.concatenate(outs, axis=-1), fcw) + fcb
    h1 = _ln(hm + h, g1, bg1)
    h2 = _dot(_lrelu01(_dot(h1, l1w) + l1b), l2w) + l2b
    return _ln(h2 + h1, g2, bg2)


def _tc_mid_body(acc, denp, h, xl1, xr1, xl2, xr2, att, bias1, bias2,
                 fcw, fcb, g1, bg1, l1w, l1b, l2w, l2b, g2, bg2,
                 nwl1, nbl1, nwr1, nbr1, nwl2, nbl2, nwr2, nbr2,
                 h_o, xl1_o, xr1_o, xl2_o, xr2_o):
    hn = _combine_convs(acc[...], denp[...], h[...], xl1[...], xr1[...],
                        xl2[...], xr2[...], att[...], bias1[...], bias2[...],
                        fcw[...], fcb[...], g1[...], bg1[...], l1w[...],
                        l1b[...], l2w[...], l2b[...], g2[...], bg2[...])
    h_o[...] = hn
    xl1_o[...] = _dot(hn, nwl1[...]) + nbl1[...]
    xr1_o[...] = _dot(hn, nwr1[...]) + nbr1[...]
    xl2_o[...] = _dot(hn, nwl2[...]) + nbl2[...]
    xr2_o[...] = _dot(hn, nwr2[...]) + nbr2[...]


def _tc_last_body(acc, denp, h, xl1, xr1, xl2, xr2, att, bias1, bias2,
                  fcw, fcb, g1, bg1, l1w, l1b, l2w, l2b, g2, bg2,
                  f1w, f1b, f2w, f2b, f3w, f3b, out_o):
    hn = _combine_convs(acc[...], denp[...], h[...], xl1[...], xr1[...],
                        xl2[...], xr2[...], att[...], bias1[...], bias2[...],
                        fcw[...], fcb[...], g1[...], bg1[...], l1w[...],
                        l1b[...], l2w[...], l2b[...], g2[...], bg2[...])
    y = _dot(_lrelu01(_dot(hn, f1w[...]) + f1b[...]), f2w[...]) + f2b[...]
    out_o[...] = _dot(_lrelu01(y), f3w[...]) + f3b[...]


_TB = 1024
_TG = (_N + _TB - 1) // _TB


def _shape_n():
    return jax.ShapeDtypeStruct((_N, _HID), jnp.float32)


def _rows_spec():
    return pl.BlockSpec((_TB, _HID), lambda i: (i, 0))


def _full_spec(shape):
    nd = len(shape)
    return pl.BlockSpec(shape, lambda i, nd=nd: (0,) * nd)


_tc_embed = pl.pallas_call(
    _tc_embed_body,
    grid=(_TG,),
    in_specs=[pl.BlockSpec((_TB, 128), lambda i: (i, 0))]
    + [_full_spec(s) for s in ((128, _HID), (1, _HID), (_HID, _HID),
                               (1, _HID))]
    + [_full_spec(s) for s in ((_HID, _HID), (1, _HID)) * 4],
    out_specs=tuple(_rows_spec() for _ in range(5)),
    out_shape=tuple(_shape_n() for _ in range(5)),
)

_mid_common_specs = (
    [pl.BlockSpec((4, _TB, _HID), lambda i: (0, i, 0)),
     pl.BlockSpec((2, _NW, _TB), lambda i: (0, 0, i))]
    + [_rows_spec() for _ in range(5)]
    + [_full_spec((2, _HID)), _full_spec((1, _HID)), _full_spec((1, _HID)),
       _full_spec((2 * _HID, _HID)), _full_spec((1, _HID)),
       _full_spec((1, _HID)), _full_spec((1, _HID)),
       _full_spec((_HID, _HID)), _full_spec((1, _HID)),
       _full_spec((_HID, _HID)), _full_spec((1, _HID)),
       _full_spec((1, _HID)), _full_spec((1, _HID))]
)

_tc_mid = pl.pallas_call(
    _tc_mid_body,
    grid=(_TG,),
    in_specs=_mid_common_specs
    + [_full_spec(s) for s in ((_HID, _HID), (1, _HID)) * 4],
    out_specs=tuple(_rows_spec() for _ in range(5)),
    out_shape=tuple(_shape_n() for _ in range(5)),
)

_tc_last = pl.pallas_call(
    _tc_last_body,
    grid=(_TG,),
    in_specs=_mid_common_specs
    + [_full_spec(s) for s in ((_HID, _HID), (1, _HID)) * 3],
    out_specs=_rows_spec(),
    out_shape=_shape_n(),
)


def _r2(b):
    return b.reshape(1, -1)


def _edge3d(v):
    return v.astype(jnp.int32).reshape(_NCH, _ROWS, _IDXW)


def kernel(x, edge_index, global_edge_index, params):
    src1 = _edge3d(edge_index[0])
    dst1 = _edge3d(edge_index[1])
    src2 = _edge3d(global_edge_index[0])
    dst2 = _edge3d(global_edge_index[1])

    p = params
    emb = p["embed_fc"]

    def gatw(c):
        return (c["lin_l"]["W"], _r2(c["lin_l"]["b"]),
                c["lin_r"]["W"], _r2(c["lin_r"]["b"]))

    c0 = p["convs"][0]
    h, xl1, xr1, xl2, xr2 = _tc_embed(
        x, emb["l1"]["W"], _r2(emb["l1"]["b"]), emb["l2"]["W"],
        _r2(emb["l2"]["b"]),
        *gatw(c0["conv1"]), *gatw(c0["conv2"]))

    for i in range(5):
        conv = p["convs"][i]
        att = jnp.concatenate([conv["conv1"]["att"], conv["conv2"]["att"]], 0)
        acc, denp = _sc_layer(xl1, xr1, xl2, xr2, att.reshape(2 * _HID),
                              src1, dst1, src2, dst2)
        acc = acc.reshape(4, _N, _HID)
        denp = denp.reshape(2, _NW, _N)
        fcs = p["fcs"][i]
        common = (acc, denp, h, xl1, xr1, xl2, xr2, att,
                  _r2(conv["conv1"]["bias"]), _r2(conv["conv2"]["bias"]),
                  conv["fc"]["W"], _r2(conv["fc"]["b"]),
                  _r2(p["gatnorms"][i]["g"]), _r2(p["gatnorms"][i]["b"]),
                  fcs["l1"]["W"], _r2(fcs["l1"]["b"]),
                  fcs["l2"]["W"], _r2(fcs["l2"]["b"]),
                  _r2(p["fcnorms"][i]["g"]), _r2(p["fcnorms"][i]["b"]))
        if i < 4:
            cn = p["convs"][i + 1]
            h, xl1, xr1, xl2, xr2 = _tc_mid(
                *common, *gatw(cn["conv1"]), *gatw(cn["conv2"]))
        else:
            f = p["fc_final"]
            out = _tc_last(
                *common, f["l1"]["W"], _r2(f["l1"]["b"]),
                f["l2"]["W"], _r2(f["l2"]["b"]),
                f["l3"]["W"], _r2(f["l3"]["b"]))
    return out


# speedup vs baseline: 47.9302x; 1.0948x over previous
import functools

import jax
import jax.numpy as jnp
from jax import lax
from jax.experimental import pallas as pl
from jax.experimental.pallas import tpu as pltpu
from jax.experimental.pallas import tpu_sc as plsc

_N = 10000
_E = 320000
_HID = 16

_IDXW = 128
_CHUNK = 512
_ROWS = _CHUNK // _IDXW
_NCH = _E // _CHUNK
_NW = 32
_NSUB = 16
_NITER = (_NCH + 2 * _NW - 1) // (2 * _NW)
_ACCW = 32
_ZPS = 624


def _lrelu01(v):
    return jnp.maximum(v, 0.01 * v)


def _lrelu02(v):
    return jnp.maximum(v, 0.2 * v)


def _ln(v, g, b):
    mu = jnp.mean(v, -1, keepdims=True)
    d = v - mu
    var = jnp.mean(d * d, -1, keepdims=True)
    return d * lax.rsqrt(var + 1e-5) * g + b


def _dot(a, w):
    return jnp.dot(a, w, preferred_element_type=jnp.float32)


def _sc_layer_body(xl1_h, xr1_h, xl2_h, xr2_h, att_h,
                   src1_h, dst1_h, src2_h, dst2_h, out_h, den_h,
                   attv, srcv, dstv, scat, gxl, gxr, outb, wv, dent,
                   acc1, acc2, sems):
    cid = lax.axis_index("c")
    sid = lax.axis_index("s")
    wid = sid * 2 + cid

    pltpu.sync_copy(att_h, attv)

    zero16 = jnp.zeros((16,), jnp.float32)

    def _zero_row(i, carry):
        for b in range(2):
            outb[b, i, pl.ds(0, 16)] = zero16
        return carry

    lax.fori_loop(0, _CHUNK, _zero_row, 0)

    def _zero_den(i, carry):
        dent[pl.ds(i * 16, 16)] = zero16
        return carry

    zbase = sid * _ZPS
    for acc in (acc1, acc2):
        pltpu.sync_copy(outb.at[0, pl.ds(0, 512)], acc.at[pl.ds(zbase, 512)])
        pltpu.sync_copy(outb.at[0, pl.ds(0, 112)],
                        acc.at[pl.ds(zbase + 512, 112)])

        @pl.when(sid == 0)
        def _(acc=acc):
            pltpu.sync_copy(outb.at[0, pl.ds(0, _N - _NSUB * _ZPS)],
                            acc.at[pl.ds(_NSUB * _ZPS, _N - _NSUB * _ZPS)])
    plsc.subcore_barrier()

    iota16 = lax.broadcasted_iota(jnp.int32, (16,), 0)
    colids = [jnp.full((16,), f, jnp.int32) for f in range(_HID + 1)]
    semI = (sems[0], sems[1])
    semG = (sems[2], sems[3])
    semS = (sems[4], sems[5])

    convs = ((xl1_h, xr1_h, src1_h, dst1_h, acc1, 0),
             (xl2_h, xr2_h, src2_h, dst2_h, acc2, 1))
    for (xl_h, xr_h, src_h, dst_h, acc, s) in convs:
        att_row = attv[pl.ds(s * _HID, _HID)]
        att_s = [att_row[f] for f in range(_HID)]
        lax.fori_loop(0, _N // 16, _zero_den, 0)

        def fire_idx(c, b, src_h=src_h, dst_h=dst_h):
            pltpu.async_copy(src_h.at[c], srcv.at[b], semI[b])
            pltpu.async_copy(dst_h.at[c], dstv.at[b], semI[b])

        def drain_idx(c, b, src_h=src_h, dst_h=dst_h):
            pltpu.make_async_copy(src_h.at[c], srcv.at[b], semI[b]).wait()
            pltpu.make_async_copy(dst_h.at[c], dstv.at[b], semI[b]).wait()

        def fire_gathers(b, xl_h=xl_h, xr_h=xr_h):
            for r in range(_ROWS):
                pltpu.async_copy(xl_h.at[srcv.at[b, r]],
                                 gxl.at[b, pl.ds(r * _IDXW, _IDXW)], semG[b])
                pltpu.async_copy(xr_h.at[dstv.at[b, r]],
                                 gxr.at[b, pl.ds(r * _IDXW, _IDXW)], semG[b])

        def drain_gathers(b, xl_h=xl_h):
            pltpu.make_async_copy(xl_h.at[pl.ds(0, _CHUNK)],
                                  gxl.at[b], semG[b]).wait()
            pltpu.make_async_copy(xl_h.at[pl.ds(0, _CHUNK)],
                                  gxr.at[b], semG[b]).wait()

        def copy_scat(b):
            for r in range(_ROWS):
                for j in range(_IDXW // 16):
                    scat[b, r, pl.ds(j * 16, 16)] = \
                        dstv[b, r, pl.ds(j * 16, 16)]

        def fire_scatters(b, acc=acc):
            for r in range(_ROWS):
                pltpu.async_copy(outb.at[b, pl.ds(r * _IDXW, _IDXW)],
                                 acc.at[scat.at[b, r]], semS[b], add=True)

        def drain_scatters(b, acc=acc):
            for r in range(_ROWS):
                pltpu.make_async_copy(outb.at[b, pl.ds(r * _IDXW, _IDXW)],
                                      acc.at[scat.at[b, r]],
                                      semS[b]).wait()

        def compute(b, att_s=att_s):
            def _group(gg, carry):
                rix = gg * 16 + iota16
                xlcols = []
                terms = []
                for f in range(_HID):
                    a = plsc.load_gather(gxl.at[b], [rix, colids[f]])
                    z = plsc.load_gather(gxr.at[b], [rix, colids[f]])
                    terms.append(att_s[f] * _lrelu02(a + z))
                    xlcols.append(a)
                while len(terms) > 1:
                    terms = [terms[k] + terms[k + 1]
                             for k in range(0, len(terms), 2)]
                w = jnp.exp(terms[0])
                for f in range(_HID):
                    plsc.store_scatter(outb.at[b], [rix, colids[f]],
                                       w * xlcols[f])
                wv[b, pl.ds(gg * 16, 16)] = w
                return carry

            lax.fori_loop(0, _CHUNK // 16, _group, 0)

            def _den(gg, carry):
                dvec = dstv[b, gg // 8, pl.ds((gg % 8) * 16, 16)]
                plsc.addupdate_scatter(dent, [dvec], wv[b, pl.ds(gg * 16, 16)])
                return carry

            lax.fori_loop(0, _CHUNK // 16, _den, 0)

        fire_idx(wid, 0)
        drain_idx(wid, 0)
        fire_gathers(0)
        fire_idx(wid + _NW, 1)

        def _iter(t, carry):
            k0 = 2 * t
            c0 = wid + k0 * _NW
            c1 = c0 + _NW
            c2 = c1 + _NW
            c3 = c2 + _NW

            @pl.when(c0 < _NCH)
            def _():
                drain_gathers(0)

            @pl.when(c1 < _NCH)
            def _():
                drain_idx(c1, 1)
                fire_gathers(1)

            @pl.when(c0 < _NCH)
            def _():
                @pl.when(c0 >= 2 * _NW)
                def _():
                    drain_scatters(0)
                copy_scat(0)
                compute(0)
                fire_scatters(0)

                @pl.when(c2 < _NCH)
                def _():
                    fire_idx(c2, 0)

            @pl.when(c1 < _NCH)
            def _():
                drain_gathers(1)

                @pl.when(c2 < _NCH)
                def _():
                    drain_idx(c2, 0)
                    fire_gathers(0)

                @pl.when(c1 >= 2 * _NW)
                def _():
                    drain_scatters(1)
                copy_scat(1)
                compute(1)
                fire_scatters(1)

                @pl.when(c3 < _NCH)
                def _():
                    fire_idx(c3, 1)

            return carry

        lax.fori_loop(0, _NITER, _iter, 0)
        drain_scatters(0)
        drain_scatters(1)
        pltpu.sync_copy(dent, den_h.at[s * _NW + wid])

    plsc.subcore_barrier()

    for s, acc in ((0, acc1), (1, acc2)):
        off = (s * 2 + cid) * _N + sid * _ZPS
        pltpu.sync_copy(acc.at[pl.ds(sid * _ZPS, _ZPS)],
                        out_h.at[pl.ds(off, _ZPS)])

        @pl.when(sid == 0)
        def _(acc=acc, s=s):
            tail = _N - _NSUB * _ZPS
            pltpu.sync_copy(acc.at[pl.ds(_NSUB * _ZPS, tail)],
                            out_h.at[pl.ds((s * 2 + cid) * _N
                                           + _NSUB * _ZPS, tail)])


_sc_layer = functools.partial(
    pl.kernel,
    out_type=(jax.ShapeDtypeStruct((4 * _N, _HID), jnp.float32),
              jax.ShapeDtypeStruct((2 * _NW, _N), jnp.float32)),
    mesh=plsc.VectorSubcoreMesh(core_axis_name="c", subcore_axis_name="s"),
    compiler_params=pltpu.CompilerParams(needs_layout_passes=False,
                                         use_tc_tiling_on_sc=False),
    scratch_types=[
        pltpu.VMEM((2 * _HID,), jnp.float32),
        pltpu.VMEM((2, _ROWS, _IDXW), jnp.int32),
        pltpu.VMEM((2, _ROWS, _IDXW), jnp.int32),
        pltpu.VMEM((2, _ROWS, _IDXW), jnp.int32),
        pltpu.VMEM((2, _CHUNK, _HID), jnp.float32),
        pltpu.VMEM((2, _CHUNK, _HID), jnp.float32),
        pltpu.VMEM((2, _CHUNK, _HID), jnp.float32),
        pltpu.VMEM((2, _CHUNK), jnp.float32),
        pltpu.VMEM((_N,), jnp.float32),
        pltpu.VMEM_SHARED((_N, _HID), jnp.float32),
        pltpu.VMEM_SHARED((_N, _HID), jnp.float32),
        [pltpu.SemaphoreType.DMA] * 6,
    ],
)(_sc_layer_body)


def _tc_embed_body(x, w1, b1, w2, b2,
                   wl1, bl1, wr1, br1, wl2, bl2, wr2, br2,
                   h_o, xl1_o, xr1_o, xl2_o, xr2_o):
    h = _dot(_lrelu01(_dot(x[...], w1[...]) + b1[...]), w2[...]) + b2[...]
    h_o[...] = h
    xl1_o[...] = _dot(h, wl1[...]) + bl1[...]
    xr1_o[...] = _dot(h, wr1[...]) + br1[...]
    xl2_o[...] = _dot(h, wl2[...]) + bl2[...]
    xr2_o[...] = _dot(h, wr2[...]) + br2[...]


def _combine_convs(acc, denp, h, xl1, xr1, xl2, xr2, att, bias1, bias2,
                   fcw, fcb, g1, bg1, l1w, l1b, l2w, l2b, g2, bg2):
    outs = []
    for s, (xl, xr, bias) in enumerate(((xl1, xr1, bias1),
                                        (xl2, xr2, bias2))):
        wself = jnp.exp(jnp.sum(_lrelu02(xl + xr) * att[s:s + 1, :],
                                axis=-1, keepdims=True))
        num = acc[2 * s] + acc[2 * s + 1] + wself * xl
        den = jnp.sum(denp[s], axis=0)[:, None] + wself
        outs.append(num / (den + 1e-16) + bias)
    hm = _dot(jnp.concatenate(outs, axis=-1), fcw) + fcb
    h1 = _ln(hm + h, g1, bg1)
    h2 = _dot(_lrelu01(_dot(h1, l1w) + l1b), l2w) + l2b
    return _ln(h2 + h1, g2, bg2)


def _tc_mid_body(acc, denp, h, xl1, xr1, xl2, xr2, att, bias1, bias2,
                 fcw, fcb, g1, bg1, l1w, l1b, l2w, l2b, g2, bg2,
                 nwl1, nbl1, nwr1, nbr1, nwl2, nbl2, nwr2, nbr2,
                 h_o, xl1_o, xr1_o, xl2_o, xr2_o):
    hn = _combine_convs(acc[...], denp[...], h[...], xl1[...], xr1[...],
                        xl2[...], xr2[...], att[...], bias1[...], bias2[...],
                        fcw[...], fcb[...], g1[...], bg1[...], l1w[...],
                        l1b[...], l2w[...], l2b[...], g2[...], bg2[...])
    h_o[...] = hn
    xl1_o[...] = _dot(hn, nwl1[...]) + nbl1[...]
    xr1_o[...] = _dot(hn, nwr1[...]) + nbr1[...]
    xl2_o[...] = _dot(hn, nwl2[...]) + nbl2[...]
    xr2_o[...] = _dot(hn, nwr2[...]) + nbr2[...]


def _tc_last_body(acc, denp, h, xl1, xr1, xl2, xr2, att, bias1, bias2,
                  fcw, fcb, g1, bg1, l1w, l1b, l2w, l2b, g2, bg2,
                  f1w, f1b, f2w, f2b, f3w, f3b, out_o):
    hn = _combine_convs(acc[...], denp[...], h[...], xl1[...], xr1[...],
                        xl2[...], xr2[...], att[...], bias1[...], bias2[...],
                        fcw[...], fcb[...], g1[...], bg1[...], l1w[...],
                        l1b[...], l2w[...], l2b[...], g2[...], bg2[...])
    y = _dot(_lrelu01(_dot(hn, f1w[...]) + f1b[...]), f2w[...]) + f2b[...]
    out_o[...] = _dot(_lrelu01(y), f3w[...]) + f3b[...]


_TB = 1024
_TG = (_N + _TB - 1) // _TB


def _shape_n():
    return jax.ShapeDtypeStruct((_N, _HID), jnp.float32)


def _rows_spec():
    return pl.BlockSpec((_TB, _HID), lambda i: (i, 0))


def _full_spec(shape):
    nd = len(shape)
    return pl.BlockSpec(shape, lambda i, nd=nd: (0,) * nd)


_tc_embed = pl.pallas_call(
    _tc_embed_body,
    grid=(_TG,),
    in_specs=[pl.BlockSpec((_TB, 128), lambda i: (i, 0))]
    + [_full_spec(s) for s in ((128, _HID), (1, _HID), (_HID, _HID),
                               (1, _HID))]
    + [_full_spec(s) for s in ((_HID, _HID), (1, _HID)) * 4],
    out_specs=tuple(_rows_spec() for _ in range(5)),
    out_shape=tuple(_shape_n() for _ in range(5)),
)

_mid_common_specs = (
    [pl.BlockSpec((4, _TB, _HID), lambda i: (0, i, 0)),
     pl.BlockSpec((2, _NW, _TB), lambda i: (0, 0, i))]
    + [_rows_spec() for _ in range(5)]
    + [_full_spec((2, _HID)), _full_spec((1, _HID)), _full_spec((1, _HID)),
       _full_spec((2 * _HID, _HID)), _full_spec((1, _HID)),
       _full_spec((1, _HID)), _full_spec((1, _HID)),
       _full_spec((_HID, _HID)), _full_spec((1, _HID)),
       _full_spec((_HID, _HID)), _full_spec((1, _HID)),
       _full_spec((1, _HID)), _full_spec((1, _HID))]
)

_tc_mid = pl.pallas_call(
    _tc_mid_body,
    grid=(_TG,),
    in_specs=_mid_common_specs
    + [_full_spec(s) for s in ((_HID, _HID), (1, _HID)) * 4],
    out_specs=tuple(_rows_spec() for _ in range(5)),
    out_shape=tuple(_shape_n() for _ in range(5)),
)

_tc_last = pl.pallas_call(
    _tc_last_body,
    grid=(_TG,),
    in_specs=_mid_common_specs
    + [_full_spec(s) for s in ((_HID, _HID), (1, _HID)) * 3],
    out_specs=_rows_spec(),
    out_shape=_shape_n(),
)


def _r2(b):
    return b.reshape(1, -1)


def _edge3d(v):
    return v.astype(jnp.int32).reshape(_NCH, _ROWS, _IDXW)


def kernel(x, edge_index, global_edge_index, params):
    src1 = _edge3d(edge_index[0])
    dst1 = _edge3d(edge_index[1])
    src2 = _edge3d(global_edge_index[0])
    dst2 = _edge3d(global_edge_index[1])

    p = params
    emb = p["embed_fc"]

    def gatw(c):
        return (c["lin_l"]["W"], _r2(c["lin_l"]["b"]),
                c["lin_r"]["W"], _r2(c["lin_r"]["b"]))

    c0 = p["convs"][0]
    h, xl1, xr1, xl2, xr2 = _tc_embed(
        x, emb["l1"]["W"], _r2(emb["l1"]["b"]), emb["l2"]["W"],
        _r2(emb["l2"]["b"]),
        *gatw(c0["conv1"]), *gatw(c0["conv2"]))

    for i in range(5):
        conv = p["convs"][i]
        att = jnp.concatenate([conv["conv1"]["att"], conv["conv2"]["att"]], 0)
        acc, denp = _sc_layer(xl1, xr1, xl2, xr2, att.reshape(2 * _HID),
                              src1, dst1, src2, dst2)
        acc = acc.reshape(4, _N, _HID)
        denp = denp.reshape(2, _NW, _N)
        fcs = p["fcs"][i]
        common = (acc, denp, h, xl1, xr1, xl2, xr2, att,
                  _r2(conv["conv1"]["bias"]), _r2(conv["conv2"]["bias"]),
                  conv["fc"]["W"], _r2(conv["fc"]["b"]),
                  _r2(p["gatnorms"][i]["g"]), _r2(p["gatnorms"][i]["b"]),
                  fcs["l1"]["W"], _r2(fcs["l1"]["b"]),
                  fcs["l2"]["W"], _r2(fcs["l2"]["b"]),
                  _r2(p["fcnorms"][i]["g"]), _r2(p["fcnorms"][i]["b"]))
        if i < 4:
            cn = p["convs"][i + 1]
            h, xl1, xr1, xl2, xr2 = _tc_mid(
                *common, *gatw(cn["conv1"]), *gatw(cn["conv2"]))
        else:
            f = p["fc_final"]
            out = _tc_last(
                *common, f["l1"]["W"], _r2(f["l1"]["b"]),
                f["l2"]["W"], _r2(f["l2"]["b"]),
                f["l3"]["W"], _r2(f["l3"]["b"]))
    return out
